# baseline (speedup 1.0000x reference)
import functools

import jax
import jax.numpy as jnp
from jax import lax
from jax.experimental import pallas as pl
from jax.experimental.pallas import tpu as pltpu
from jax.experimental.pallas import tpu_sc as plsc

EMB = 128
NROWS = 65
N = 100000
CHUNK = 80
NCHUNKS = N // CHUNK
NW = 32
GSZ = 3
NSET = 2
NFULL = 39
NG = NFULL // GSZ
NDG = NG // 2


def _body(w_hbm, z_hbm, out_hbm, w_sh, idx_v, *scratch):
    rows = scratch[0:NSET * GSZ]
    gsem = scratch[NSET * GSZ:2 * NSET * GSZ]
    osem = scratch[2 * NSET * GSZ:3 * NSET * GSZ]
    isem = scratch[3 * NSET * GSZ]
    wid = lax.axis_index("s") * 2 + lax.axis_index("c")
    has_extra = wid < NCHUNKS - NFULL * NW

    def chunk_base(t):
        return (wid + t * NW) * CHUNK

    def idx_desc(t):
        return pltpu.make_async_copy(
            z_hbm.at[pl.ds(chunk_base(t), CHUNK)], idx_v.at[t], isem)

    @pl.when(lax.axis_index("s") == 0)
    def _():
        pltpu.sync_copy(w_hbm, w_sh.at[pl.ds(1, NROWS - 1)])

    for t in range(NFULL):
        idx_desc(t).start()

    @pl.when(has_extra)
    def _():
        idx_desc(NFULL).start()

    for t in range(NFULL):
        idx_desc(t).wait()

    @pl.when(has_extra)
    def _():
        idx_desc(NFULL).wait()

    plsc.subcore_barrier()

    def gather_desc(t, u):
        return pltpu.make_async_copy(w_sh.at[idx_v.at[t]], rows[u], gsem[u])

    def out_desc(t, u):
        return pltpu.make_async_copy(
            rows[u], out_hbm.at[pl.ds(chunk_base(t), CHUNK)], osem[u])

    for u in range(NSET * GSZ):
        gather_desc(u, u).start()

    def dgroup(gg, carry):
        t0 = gg * NSET * GSZ
        for s in range(NSET):
            for b in range(GSZ):
                u = s * GSZ + b
                gather_desc(t0 + u, u).wait()
                out_desc(t0 + u, u).start()
        for s in range(NSET):
            for b in range(GSZ):
                u = s * GSZ + b
                t = t0 + NSET * GSZ + u
                @pl.when(t < NFULL)
                def _():
                    out_desc(t - NSET * GSZ, u).wait()
                    gather_desc(t, u).start()
        return carry

    lax.fori_loop(0, NDG, dgroup, 0)

    t0 = NDG * NSET * GSZ
    for b in range(GSZ):
        gather_desc(t0 + b, b).wait()
        out_desc(t0 + b, b).start()
        out_desc(t0 + b, b).wait()
    for b in range(GSZ):
        u = GSZ + b
        out_desc(t0 - GSZ + b, u).wait()

    @pl.when(has_extra)
    def _():
        gather_desc(NFULL, GSZ).start()
        gather_desc(NFULL, GSZ).wait()
        out_desc(NFULL, GSZ).start()
        out_desc(NFULL, GSZ).wait()


def kernel(Z, W):
    mesh = plsc.VectorSubcoreMesh(core_axis_name="c", subcore_axis_name="s")
    k = functools.partial(
        pl.kernel,
        mesh=mesh,
        compiler_params=pltpu.CompilerParams(skip_device_barrier=True),
        out_type=jax.ShapeDtypeStruct((N, EMB), jnp.float32),
        scratch_types=(
            [pltpu.VMEM_SHARED((NROWS, EMB), jnp.float32),
             pltpu.VMEM((NFULL + 1, CHUNK), jnp.int32)]
            + [pltpu.VMEM((CHUNK, EMB), jnp.float32)
               for _ in range(NSET * GSZ)]
            + [pltpu.SemaphoreType.DMA for _ in range(2 * NSET * GSZ + 1)]
        ),
    )(_body)
    return k(W, Z)


# speedup vs baseline: 1.0079x; 1.0079x over previous
import functools

import jax
import jax.numpy as jnp
from jax import lax
from jax.experimental import pallas as pl
from jax.experimental.pallas import tpu as pltpu
from jax.experimental.pallas import tpu_sc as plsc

EMB = 128
NROWS = 65
N = 100000
CHUNK = 80
NCHUNKS = N // CHUNK
NW = 32
GSZ = 3
NSET = 2
NFULL = 39
NG = NFULL // GSZ
NDG = NG // 2


def _body(w_hbm, z_hbm, out_hbm, w_sh, idx_v, *scratch):
    rows = scratch[0:NSET * GSZ]
    gsem = scratch[NSET * GSZ:2 * NSET * GSZ]
    osem = scratch[2 * NSET * GSZ:3 * NSET * GSZ]
    isem = scratch[3 * NSET * GSZ]
    wid = lax.axis_index("s") * 2 + lax.axis_index("c")
    has_extra = wid < NCHUNKS - NFULL * NW

    def chunk_base(t):
        return (wid + t * NW) * CHUNK

    def idx_desc(t):
        return pltpu.make_async_copy(
            z_hbm.at[pl.ds(chunk_base(t), CHUNK)], idx_v.at[t], isem)

    @pl.when(lax.axis_index("s") == 0)
    def _():
        pltpu.sync_copy(w_hbm, w_sh.at[pl.ds(1, NROWS - 1)])

    def idx_start(t, carry):
        idx_desc(t).start()
        return carry

    def idx_wait(t, carry):
        idx_desc(t).wait()
        return carry

    n_idx = NFULL + jnp.where(has_extra, 1, 0)
    lax.fori_loop(0, n_idx, idx_start, 0)
    lax.fori_loop(0, n_idx, idx_wait, 0)

    plsc.subcore_barrier()

    def gather_desc(t, u):
        return pltpu.make_async_copy(w_sh.at[idx_v.at[t]], rows[u], gsem[u])

    def out_desc(t, u):
        return pltpu.make_async_copy(
            rows[u], out_hbm.at[pl.ds(chunk_base(t), CHUNK)], osem[u])

    for u in range(NSET * GSZ):
        gather_desc(u, u).start()

    def dgroup(gg, carry):
        t0 = gg * NSET * GSZ
        for s in range(NSET):
            for b in range(GSZ):
                u = s * GSZ + b
                gather_desc(t0 + u, u).wait()
                out_desc(t0 + u, u).start()
        for s in range(NSET):
            for b in range(GSZ):
                u = s * GSZ + b
                t = t0 + NSET * GSZ + u
                @pl.when(t < NFULL)
                def _():
                    out_desc(t - NSET * GSZ, u).wait()
                    gather_desc(t, u).start()
        return carry

    lax.fori_loop(0, NDG, dgroup, 0)

    t0 = NDG * NSET * GSZ
    for b in range(GSZ):
        gather_desc(t0 + b, b).wait()
        out_desc(t0 + b, b).start()
        out_desc(t0 + b, b).wait()
    for b in range(GSZ):
        u = GSZ + b
        out_desc(t0 - GSZ + b, u).wait()

    @pl.when(has_extra)
    def _():
        gather_desc(NFULL, GSZ).start()
        gather_desc(NFULL, GSZ).wait()
        out_desc(NFULL, GSZ).start()
        out_desc(NFULL, GSZ).wait()


def kernel(Z, W):
    mesh = plsc.VectorSubcoreMesh(core_axis_name="c", subcore_axis_name="s")
    k = functools.partial(
        pl.kernel,
        mesh=mesh,
        out_type=jax.ShapeDtypeStruct((N, EMB), jnp.float32),
        scratch_types=(
            [pltpu.VMEM_SHARED((NROWS, EMB), jnp.float32),
             pltpu.VMEM((NFULL + 1, CHUNK), jnp.int32)]
            + [pltpu.VMEM((CHUNK, EMB), jnp.float32)
               for _ in range(NSET * GSZ)]
            + [pltpu.SemaphoreType.DMA for _ in range(2 * NSET * GSZ + 1)]
        ),
    )(_body)
    return k(W, Z)


# speedup vs baseline: 1.0414x; 1.0333x over previous
import functools

import jax
import jax.numpy as jnp
from jax import lax
from jax.experimental import pallas as pl
from jax.experimental.pallas import tpu as pltpu
from jax.experimental.pallas import tpu_sc as plsc

EMB = 128
NROWS = 65
N = 100000
CHUNK = 80
NCHUNKS = N // CHUNK
NW = 32
NB = 6
LEAD = 3
NFULL = 39


def _body(w_hbm, z_hbm, out_hbm, w_sh, idx_v, *scratch):
    rows = scratch[0:NB]
    gsem = scratch[NB:2 * NB]
    osem = scratch[2 * NB:3 * NB]
    isem = scratch[3 * NB]
    wid = lax.axis_index("s") * 2 + lax.axis_index("c")
    has_extra = wid < NCHUNKS - NFULL * NW

    def chunk_base(t):
        return (wid + t * NW) * CHUNK

    def idx_desc(t):
        return pltpu.make_async_copy(
            z_hbm.at[pl.ds(chunk_base(t), CHUNK)], idx_v.at[t], isem)

    @pl.when(lax.axis_index("s") == 0)
    def _():
        pltpu.sync_copy(w_hbm, w_sh.at[pl.ds(1, NROWS - 1)])

    n_idx = NFULL + jnp.where(has_extra, 1, 0)
    lax.fori_loop(0, n_idx, lambda t, c: (idx_desc(t).start(), c)[1], 0)
    lax.fori_loop(0, n_idx, lambda t, c: (idx_desc(t).wait(), c)[1], 0)

    plsc.subcore_barrier()

    def gather_desc(t, u):
        return pltpu.make_async_copy(w_sh.at[idx_v.at[t]], rows[u], gsem[u])

    def out_desc(t, u):
        return pltpu.make_async_copy(
            rows[u], out_hbm.at[pl.ds(chunk_base(t), CHUNK)], osem[u])

    for u in range(LEAD):
        gather_desc(u, u).start()

    for i in range(NB):
        gather_desc(i, i).wait()
        out_desc(i, i).start()
        if i < LEAD:
            gather_desc(i + LEAD, i + LEAD).start()
        else:
            out_desc(i - LEAD, i - LEAD).wait()
            gather_desc(i + LEAD, i - LEAD).start()

    def steady(gg, carry):
        t0 = gg * NB
        for i in range(NB):
            t = t0 + i
            v = (i + LEAD) % NB
            gather_desc(t, i).wait()
            out_desc(t, i).start()
            out_desc(t - LEAD, v).wait()
            gather_desc(t + LEAD, v).start()
        return carry

    lax.fori_loop(1, NFULL // NB, steady, 0)

    for i in range(LEAD):
        gather_desc(NFULL - LEAD + i, i).wait()
        out_desc(NFULL - LEAD + i, i).start()
    for i in range(LEAD):
        out_desc(NFULL - 2 * LEAD + i, LEAD + i).wait()

    @pl.when(has_extra)
    def _():
        gather_desc(NFULL, LEAD).start()
        gather_desc(NFULL, LEAD).wait()
        out_desc(NFULL, LEAD).start()

    for i in range(LEAD):
        out_desc(NFULL - LEAD + i, i).wait()

    @pl.when(has_extra)
    def _():
        out_desc(NFULL, LEAD).wait()


def kernel(Z, W):
    mesh = plsc.VectorSubcoreMesh(core_axis_name="c", subcore_axis_name="s")
    k = functools.partial(
        pl.kernel,
        mesh=mesh,
        out_type=jax.ShapeDtypeStruct((N, EMB), jnp.float32),
        scratch_types=(
            [pltpu.VMEM_SHARED((NROWS, EMB), jnp.float32),
             pltpu.VMEM((NFULL + 1, CHUNK), jnp.int32)]
            + [pltpu.VMEM((CHUNK, EMB), jnp.float32) for _ in range(NB)]
            + [pltpu.SemaphoreType.DMA for _ in range(2 * NB + 1)]
        ),
    )(_body)
    return k(W, Z)
